# baseline (speedup 1.0000x reference)
import functools

import jax
import jax.numpy as jnp
from jax import lax
from jax.experimental import pallas as pl
from jax.experimental.pallas import tpu as pltpu
from jax.experimental.pallas import tpu_sc as plsc

N = 10000
E = 320000
DF = 128
H = 128
L = 4
HEADS = 4
HD = H // HEADS

NC = 2
NS = 16
LANES = 16
W = NC * NS

NE_TOT = E + N
CH = 128
CPW = 4 * (-(-NE_TOT // (W * CH * 4)))
EPW = CPW * CH
NE_PAD = W * EPW
DUMMY = N
NR = 10240
RPT = NR // (NS * CH)

NB = 400

_mesh_cache = []


def _mesh():
    if not _mesh_cache:
        _mesh_cache.append(plsc.VectorSubcoreMesh(
            core_axis_name="c", subcore_axis_name="s",
            num_cores=NC, num_subcores=NS))
    return _mesh_cache[0]


def _zero_vmem_rows(rows):
    def zrow(i, _):
        for j in range(H // LANES):
            rows[i, pl.ds(j * LANES, LANES)] = jnp.zeros((LANES,), jnp.float32)
        return 0
    lax.fori_loop(0, CH, zrow, 0)


@functools.cache
def _deg_kernel():
  kern = functools.partial(
    pl.kernel,
    out_type=jax.ShapeDtypeStruct((NC, NR), jnp.float32),
    mesh=_mesh(),
    scratch_types=[
        pltpu.VMEM((CH,), jnp.int32),
        pltpu.VMEM((CH,), jnp.float32),
        pltpu.VMEM((CH,), jnp.float32),
        pltpu.VMEM_SHARED((NR,), jnp.float32),
    ],
  )

  @kern
  def _deg_sc(dst_hbm, out_hbm, didx, ones_v, zero_v, dacc):
    c = lax.axis_index("c")
    s = lax.axis_index("s")
    w = c * NS + s
    for j in range(CH // LANES):
        ones_v[pl.ds(j * LANES, LANES)] = jnp.ones((LANES,), jnp.float32)
        zero_v[pl.ds(j * LANES, LANES)] = jnp.zeros((LANES,), jnp.float32)
    for t in range(NR // (NS * CH)):
        pltpu.sync_copy(zero_v, dacc.at[pl.ds((s * RPT + t) * CH, CH)])
    plsc.subcore_barrier()

    def body(i, _):
        base = w * EPW + i * CH
        pltpu.sync_copy(dst_hbm.at[pl.ds(base, CH)], didx)
        pltpu.sync_copy(ones_v, dacc.at[didx], add=True)
        return 0
    lax.fori_loop(0, CPW, body, 0)
    plsc.subcore_barrier()
    for t in range(RPT):
        r = (s * RPT + t) * CH
        pltpu.sync_copy(dacc.at[pl.ds(r, CH)], out_hbm.at[c, pl.ds(r, CH)])

  return _deg_sc


@functools.cache
def _edge_kernel(with_ex):
  scratch = [
      pltpu.VMEM((4, CH), jnp.int32),
      pltpu.VMEM((4, CH), jnp.int32),
      pltpu.VMEM((2, CH, H), jnp.float32),
      pltpu.VMEM((4, HEADS, CH), jnp.float32),
      pltpu.VMEM_SHARED((NR, H), jnp.float32),
      pltpu.SemaphoreType.DMA,
      pltpu.SemaphoreType.DMA,
      pltpu.SemaphoreType.DMA,
      pltpu.SemaphoreType.DMA,
      pltpu.SemaphoreType.DMA,
      pltpu.SemaphoreType.DMA,
      pltpu.SemaphoreType.DMA,
      pltpu.SemaphoreType.DMA,
  ]
  kern = functools.partial(
    pl.kernel,
    out_type=jax.ShapeDtypeStruct((NC, NR, H), jnp.float32),
    mesh=_mesh(),
    scratch_types=scratch,
  )

  def _body(hp_hbm, src_hbm, dst_hbm, ex_hbm, out_hbm,
            sidx, didx, rows, exb, accum, sem_i0, sem_i1, sem_i2, sem_i3,
            sem_g0, sem_g1, sem_s0, sem_s1):
    c = lax.axis_index("c")
    s = lax.axis_index("s")
    w = c * NS + s
    sem_i = (sem_i0, sem_i1, sem_i2, sem_i3)
    sem_g = (sem_g0, sem_g1)
    sem_s = (sem_s0, sem_s1)

    def zrow(i, _):
        for j in range(H // LANES):
            rows[0, i, pl.ds(j * LANES, LANES)] = jnp.zeros((LANES,), jnp.float32)
        return 0
    lax.fori_loop(0, CH, zrow, 0)
    for t in range(RPT):
        pltpu.sync_copy(rows.at[0], accum.at[pl.ds((s * RPT + t) * CH, CH)])
    plsc.subcore_barrier()

    def start_idx(slot, chunk):
        base = w * EPW + chunk * CH
        pltpu.async_copy(src_hbm.at[pl.ds(base, CH)], sidx.at[slot], sem_i[slot])
        pltpu.async_copy(dst_hbm.at[pl.ds(base, CH)], didx.at[slot], sem_i[slot])
        if with_ex:
            for h in range(HEADS):
                pltpu.async_copy(ex_hbm.at[h, pl.ds(base, CH)],
                                 exb.at[slot, h], sem_i[slot])

    def wait_idx(slot):
        pltpu.make_async_copy(src_hbm.at[pl.ds(0, CH)], sidx.at[slot],
                              sem_i[slot]).wait()
        pltpu.make_async_copy(dst_hbm.at[pl.ds(0, CH)], didx.at[slot],
                              sem_i[slot]).wait()
        if with_ex:
            for h in range(HEADS):
                pltpu.make_async_copy(ex_hbm.at[h, pl.ds(0, CH)],
                                      exb.at[slot, h], sem_i[slot]).wait()

    def do_mul(slot, r):
        if not with_ex:
            return

        def mul(g, _):
            exv = [exb[slot, h, pl.ds(g * LANES, LANES)] for h in range(HEADS)]
            for el in range(LANES):
                e2 = g * LANES + el
                for h in range(HEADS):
                    x = exv[h][el]
                    for k2 in range(HD // LANES):
                        off = h * HD + k2 * LANES
                        rows[r, e2, pl.ds(off, LANES)] = (
                            rows[r, e2, pl.ds(off, LANES)] * x)
            return 0
        lax.fori_loop(0, CH // LANES, mul, 0)

    def finish_chunk(slot, r):
        pltpu.make_async_copy(hp_hbm.at[sidx.at[slot]], rows.at[r],
                              sem_g[r]).wait()
        do_mul(slot, r)
        pltpu.async_copy(rows.at[r], accum.at[didx.at[slot]], sem_s[r], add=True)

    def start_gather(slot, r):
        pltpu.async_copy(hp_hbm.at[sidx.at[slot]], rows.at[r], sem_g[r])

    def drain_scatter(slot, r):
        pltpu.make_async_copy(rows.at[r], accum.at[didx.at[slot]],
                              sem_s[r]).wait()

    start_idx(0, 0)
    start_idx(1, 1)
    wait_idx(0)
    start_gather(0, 0)
    start_idx(2, 2)
    wait_idx(1)
    start_gather(1, 1)
    start_idx(3, 3)
    finish_chunk(0, 0)

    def body(t, _):
        for b in range(4):
            i = t * 4 + 2 + b
            sl = (2 + b) % 4
            r = b % 2
            wait_idx(sl)
            drain_scatter(b, r)
            start_gather(sl, r)
            start_idx(b, i + 2)
            finish_chunk((1 + b) % 4, 1 - r)
        return 0
    lax.fori_loop(0, (CPW - 4) // 4, body, 0)

    wait_idx(2)
    drain_scatter(0, 0)
    start_gather(2, 0)
    finish_chunk(1, 1)
    wait_idx(3)
    drain_scatter(1, 1)
    start_gather(3, 1)
    finish_chunk(2, 0)
    finish_chunk(3, 1)
    drain_scatter(2, 0)
    drain_scatter(3, 1)
    plsc.subcore_barrier()
    for t in range(RPT):
        rr = (s * RPT + t) * CH
        pltpu.sync_copy(accum.at[pl.ds(rr, CH)], out_hbm.at[c, pl.ds(rr, CH)])

  if with_ex:
    @kern
    def _edge_sc(hp_hbm, src_hbm, dst_hbm, ex_hbm, out_hbm, *rest):
      _body(hp_hbm, src_hbm, dst_hbm, ex_hbm, out_hbm, *rest)
  else:
    @kern
    def _edge_sc(hp_hbm, src_hbm, dst_hbm, out_hbm, *rest):
      _body(hp_hbm, src_hbm, dst_hbm, None, out_hbm, *rest)

  return _edge_sc


SPT = NR * HEADS // (NS * CH)


@functools.cache
def _att_kernel():
  kern = functools.partial(
    pl.kernel,
    out_type=(jax.ShapeDtypeStruct((HEADS, NE_PAD), jnp.float32),
              jax.ShapeDtypeStruct((NC, NR * HEADS), jnp.float32)),
    mesh=_mesh(),
    compiler_params=pltpu.CompilerParams(needs_layout_passes=False),
    scratch_types=[
        pltpu.VMEM((NR * HEADS,), jnp.float32),
        pltpu.VMEM((NR * HEADS,), jnp.float32),
        pltpu.VMEM((LANES,), jnp.float32),
        pltpu.VMEM((CH,), jnp.int32),
        pltpu.VMEM((CH,), jnp.int32),
        pltpu.VMEM((HEADS, CH), jnp.float32),
        pltpu.VMEM((HEADS, CH), jnp.int32),
        pltpu.VMEM((CH,), jnp.float32),
        pltpu.VMEM_SHARED((NR * HEADS,), jnp.float32),
    ],
  )

  @kern
  def _att_sc(as_hbm, ad_hbm, m_hbm, src_hbm, dst_hbm, ex_hbm, s_hbm,
              as_v, ad_v, m_v, sidx, didx, exb, sxb, zbuf, sacc):
    c = lax.axis_index("c")
    s = lax.axis_index("s")
    w = c * NS + s
    pltpu.sync_copy(as_hbm, as_v)
    pltpu.sync_copy(ad_hbm, ad_v)
    pltpu.sync_copy(m_hbm, m_v)
    mvec = m_v[...]
    for j in range(CH // LANES):
        zbuf[pl.ds(j * LANES, LANES)] = jnp.zeros((LANES,), jnp.float32)
    for t in range(SPT):
        pltpu.sync_copy(zbuf, sacc.at[pl.ds((s * SPT + t) * CH, CH)])
    plsc.subcore_barrier()

    def body(i, _):
        base = w * EPW + i * CH
        pltpu.sync_copy(src_hbm.at[pl.ds(base, CH)], sidx)
        pltpu.sync_copy(dst_hbm.at[pl.ds(base, CH)], didx)
        for g in range(CH // LANES):
            sv = sidx[pl.ds(g * LANES, LANES)] * HEADS
            dv = didx[pl.ds(g * LANES, LANES)] * HEADS
            for h in range(HEADS):
                av = plsc.load_gather(as_v, [sv + h])
                bv = plsc.load_gather(ad_v, [dv + h])
                z = av + bv
                e = jnp.where(z >= 0, z, z * 0.2) - mvec[h]
                exb[h, pl.ds(g * LANES, LANES)] = jnp.exp(e)
                sxb[h, pl.ds(g * LANES, LANES)] = dv + h
        for h in range(HEADS):
            pltpu.sync_copy(exb.at[h], ex_hbm.at[h, pl.ds(base, CH)])
            pltpu.sync_copy(exb.at[h], sacc.at[sxb.at[h]], add=True)
        return 0
    lax.fori_loop(0, CPW, body, 0)
    plsc.subcore_barrier()
    for t in range(SPT):
        r = (s * SPT + t) * CH
        pltpu.sync_copy(sacc.at[pl.ds(r, CH)], s_hbm.at[c, pl.ds(r, CH)])

  return _att_sc


def _gat_sc(attn, srcw, dstw, Wl, asl, adl, bl):
    Wcat = jnp.moveaxis(Wl, 0, 1).reshape(H, H)
    h = attn @ Wcat
    hh = h.reshape(N, HEADS, HD)
    asn = (hh * asl[None]).sum(-1)
    adn = (hh * adl[None]).sum(-1)
    M = jnp.max(asn, axis=0) + jnp.max(adn, axis=0)
    M = jnp.where(M >= 0, M, 0.2 * M)
    Mp = jnp.zeros((LANES,), jnp.float32).at[:HEADS].set(M)
    asp = jnp.zeros((NR, HEADS), jnp.float32).at[:N].set(asn).reshape(-1)
    adp = jnp.zeros((NR, HEADS), jnp.float32).at[:N].set(adn).reshape(-1)
    ex, s2 = _att_kernel()(asp, adp, Mp, srcw, dstw)
    sn = (s2[0] + s2[1]).reshape(NR, HEADS)[:N]
    hp = jnp.zeros((NR, H), jnp.float32).at[:N].set(h)
    agg2 = _edge_kernel(True)(hp, srcw, dstw, ex)
    agg = (agg2[0] + agg2[1])[:N].reshape(N, HEADS, HD)
    out = agg / (sn[:, :, None] + 1e-16) + bl[None]
    return out.reshape(N, H)


def _encoder_body(nf_ref, w1_ref, b1_ref, w2_ref, b2_ref, g_ref, bb_ref, o_ref):
    x = jnp.maximum(jnp.dot(nf_ref[...], w1_ref[...],
                            preferred_element_type=jnp.float32) + b1_ref[...], 0.0)
    x = jnp.dot(x, w2_ref[...], preferred_element_type=jnp.float32) + b2_ref[...]
    m = x.mean(-1, keepdims=True)
    v = ((x - m) ** 2).mean(-1, keepdims=True)
    o_ref[...] = (x - m) * lax.rsqrt(v + 1e-5) * g_ref[...] + bb_ref[...]


def _encoder(node_features, p):
    return pl.pallas_call(
        _encoder_body,
        grid=(N // NB,),
        in_specs=[
            pl.BlockSpec((NB, DF), lambda i: (i, 0)),
            pl.BlockSpec((DF, H), lambda i: (0, 0)),
            pl.BlockSpec((H,), lambda i: (0,)),
            pl.BlockSpec((H, H), lambda i: (0, 0)),
            pl.BlockSpec((H,), lambda i: (0,)),
            pl.BlockSpec((H,), lambda i: (0,)),
            pl.BlockSpec((H,), lambda i: (0,)),
        ],
        out_specs=pl.BlockSpec((NB, H), lambda i: (i, 0)),
        out_shape=jax.ShapeDtypeStruct((N, H), jnp.float32),
    )(node_features, p['enc_W1'], p['enc_b1'], p['enc_W2'], p['enc_b2'],
      p['enc_ln_g'], p['enc_ln_b'])


def _gat_jnp(x, src, dst, Wl, asl, adl, bl, n):
    heads = []
    for hh in range(HEADS):
        h = x @ Wl[hh]
        asrc = (h * asl[hh]).sum(-1)
        adst = (h * adl[hh]).sum(-1)
        e = jax.nn.leaky_relu(asrc[src] + adst[dst], 0.2)
        m = jax.ops.segment_max(e, dst, num_segments=n)
        ex = jnp.exp(e - m[dst])
        s = jax.ops.segment_sum(ex, dst, num_segments=n)
        alpha = ex / (s[dst] + 1e-16)
        heads.append(jax.ops.segment_sum(alpha[:, None] * h[src], dst, num_segments=n) + bl[hh])
    return jnp.concatenate(heads, axis=-1)


def kernel(node_features, edge_index, params):
    p = params
    n = N
    loop = jnp.arange(n, dtype=edge_index.dtype)
    src = jnp.concatenate([edge_index[0], loop])
    dst = jnp.concatenate([edge_index[1], loop])
    pad = jnp.full((NE_PAD - NE_TOT,), DUMMY, dtype=edge_index.dtype)
    srcw = jnp.concatenate([src, pad])
    dstw = jnp.concatenate([dst, pad])

    deg2 = _deg_kernel()(dstw)
    deg = (deg2[0] + deg2[1])[:n]
    dinv = jnp.where(deg > 0, 1.0 / jnp.sqrt(deg), 0.0)

    x = _encoder(node_features, p)

    attn = x
    for i in range(L):
        out = _gat_sc(attn, srcw, dstw, p['gat_W'][i], p['gat_asrc'][i],
                      p['gat_adst'][i], p['gat_b'][i])
        out = out @ p['proj_W'][i] + p['proj_b'][i]
        m = out.mean(-1, keepdims=True)
        v = ((out - m) ** 2).mean(-1, keepdims=True)
        out = (out - m) / jnp.sqrt(v + 1e-5) * p['ln_g'][i] + p['ln_b'][i]
        attn = attn + out

    trad = x
    for i in range(L):
        hp = jnp.zeros((NR, H), jnp.float32).at[:n].set(dinv[:, None] * (trad @ p['gcn_W'][i]))
        agg2 = _edge_kernel(False)(hp, srcw, dstw)
        agg = dinv[:, None] * (agg2[0] + agg2[1])[:n] + p['gcn_b'][i]
        trad = trad + jax.nn.relu(agg)

    combined = attn + trad
    g = jnp.concatenate([combined.mean(axis=0), combined.max(axis=0)])

    def mlp3(v, W1, b1, W2, b2, W3, b3):
        h1 = jax.nn.relu(v @ W1 + b1)
        h2 = jax.nn.relu(h1 @ W2 + b2)
        return h2 @ W3 + b3

    dec = jax.nn.sigmoid(mlp3(g, p['dec_W1'], p['dec_b1'], p['dec_W2'], p['dec_b2'], p['dec_W3'], p['dec_b3']))
    val = mlp3(g, p['val_W1'], p['val_b1'], p['val_W2'], p['val_b2'], p['val_W3'], p['val_b3'])
    temp = jax.nn.relu(g @ p['tmp_W1'] + p['tmp_b1']) @ p['tmp_W2'] + p['tmp_b2']
    safe = jax.nn.sigmoid(jax.nn.relu(g @ p['safe_W1'] + p['safe_b1']) @ p['safe_W2'] + p['safe_b2'])
    return dec, val, temp, safe


# speedup vs baseline: 1.0003x; 1.0003x over previous
import functools

import jax
import jax.numpy as jnp
from jax import lax
from jax.experimental import pallas as pl
from jax.experimental.pallas import tpu as pltpu
from jax.experimental.pallas import tpu_sc as plsc

N = 10000
E = 320000
DF = 128
H = 128
L = 4
HEADS = 4
HD = H // HEADS

NC = 2
NS = 16
LANES = 16
W = NC * NS

NE_TOT = E + N
CH = 128
CPW = 4 * (-(-NE_TOT // (W * CH * 4)))
EPW = CPW * CH
NE_PAD = W * EPW
DUMMY = N
NR = 10240
RPT = NR // (NS * CH)

NB = 400

_mesh_cache = []


def _mesh():
    if not _mesh_cache:
        _mesh_cache.append(plsc.VectorSubcoreMesh(
            core_axis_name="c", subcore_axis_name="s",
            num_cores=NC, num_subcores=NS))
    return _mesh_cache[0]


def _zero_vmem_rows(rows):
    def zrow(i, _):
        for j in range(H // LANES):
            rows[i, pl.ds(j * LANES, LANES)] = jnp.zeros((LANES,), jnp.float32)
        return 0
    lax.fori_loop(0, CH, zrow, 0)


@functools.cache
def _deg_kernel():
  kern = functools.partial(
    pl.kernel,
    out_type=jax.ShapeDtypeStruct((NC, NR), jnp.float32),
    mesh=_mesh(),
    scratch_types=[
        pltpu.VMEM((CH,), jnp.int32),
        pltpu.VMEM((CH,), jnp.float32),
        pltpu.VMEM((CH,), jnp.float32),
        pltpu.VMEM_SHARED((NR,), jnp.float32),
    ],
  )

  @kern
  def _deg_sc(dst_hbm, out_hbm, didx, ones_v, zero_v, dacc):
    c = lax.axis_index("c")
    s = lax.axis_index("s")
    w = c * NS + s
    for j in range(CH // LANES):
        ones_v[pl.ds(j * LANES, LANES)] = jnp.ones((LANES,), jnp.float32)
        zero_v[pl.ds(j * LANES, LANES)] = jnp.zeros((LANES,), jnp.float32)
    for t in range(NR // (NS * CH)):
        pltpu.sync_copy(zero_v, dacc.at[pl.ds((s * RPT + t) * CH, CH)])
    plsc.subcore_barrier()

    def body(i, _):
        base = w * EPW + i * CH
        pltpu.sync_copy(dst_hbm.at[pl.ds(base, CH)], didx)
        pltpu.sync_copy(ones_v, dacc.at[didx], add=True)
        return 0
    lax.fori_loop(0, CPW, body, 0)
    plsc.subcore_barrier()
    for t in range(RPT):
        r = (s * RPT + t) * CH
        pltpu.sync_copy(dacc.at[pl.ds(r, CH)], out_hbm.at[c, pl.ds(r, CH)])

  return _deg_sc


@functools.cache
def _edge_kernel(with_ex):
  scratch = (
      [pltpu.VMEM((CH,), jnp.int32) for _ in range(4)]
      + [pltpu.VMEM((CH,), jnp.int32) for _ in range(4)]
      + [pltpu.VMEM((CH, H), jnp.float32) for _ in range(2)]
      + [pltpu.VMEM((HEADS, CH), jnp.float32) for _ in range(4)]
      + [pltpu.VMEM_SHARED((NR, H), jnp.float32)]
      + [pltpu.SemaphoreType.DMA for _ in range(8)]
  )
  kern = functools.partial(
    pl.kernel,
    out_type=jax.ShapeDtypeStruct((NC, NR, H), jnp.float32),
    mesh=_mesh(),
    scratch_types=scratch,
  )

  def _body(hp_hbm, src_hbm, dst_hbm, ex_hbm, out_hbm, *rest):
    sidx = rest[0:4]
    didx = rest[4:8]
    rows = rest[8:10]
    exb = rest[10:14]
    accum = rest[14]
    sem_i = rest[15:19]
    sem_g = rest[19:21]
    sem_s = rest[21:23]
    c = lax.axis_index("c")
    s = lax.axis_index("s")
    w = c * NS + s

    def zrow(i, _):
        for j in range(H // LANES):
            rows[0][i, pl.ds(j * LANES, LANES)] = jnp.zeros((LANES,), jnp.float32)
        return 0
    lax.fori_loop(0, CH, zrow, 0)
    for t in range(RPT):
        pltpu.sync_copy(rows[0], accum.at[pl.ds((s * RPT + t) * CH, CH)])
    plsc.subcore_barrier()

    def start_idx(slot, chunk):
        base = w * EPW + chunk * CH
        pltpu.async_copy(src_hbm.at[pl.ds(base, CH)], sidx[slot], sem_i[slot])
        pltpu.async_copy(dst_hbm.at[pl.ds(base, CH)], didx[slot], sem_i[slot])
        if with_ex:
            for h in range(HEADS):
                pltpu.async_copy(ex_hbm.at[h, pl.ds(base, CH)],
                                 exb[slot].at[h], sem_i[slot])

    def wait_idx(slot):
        pltpu.make_async_copy(src_hbm.at[pl.ds(0, CH)], sidx[slot],
                              sem_i[slot]).wait()
        pltpu.make_async_copy(dst_hbm.at[pl.ds(0, CH)], didx[slot],
                              sem_i[slot]).wait()
        if with_ex:
            for h in range(HEADS):
                pltpu.make_async_copy(ex_hbm.at[h, pl.ds(0, CH)],
                                      exb[slot].at[h], sem_i[slot]).wait()

    def do_mul(slot, r):
        if not with_ex:
            return

        def mul(g, _):
            exv = [exb[slot][h, pl.ds(g * LANES, LANES)] for h in range(HEADS)]
            for el in range(LANES):
                e2 = g * LANES + el
                for h in range(HEADS):
                    x = exv[h][el]
                    for k2 in range(HD // LANES):
                        off = h * HD + k2 * LANES
                        rows[r][e2, pl.ds(off, LANES)] = (
                            rows[r][e2, pl.ds(off, LANES)] * x)
            return 0
        lax.fori_loop(0, CH // LANES, mul, 0)

    def finish_chunk(slot, r):
        pltpu.make_async_copy(hp_hbm.at[sidx[slot]], rows[r], sem_g[r]).wait()
        do_mul(slot, r)
        pltpu.async_copy(rows[r], accum.at[didx[slot]], sem_s[r], add=True)

    def start_gather(slot, r):
        pltpu.async_copy(hp_hbm.at[sidx[slot]], rows[r], sem_g[r])

    def drain_scatter(slot, r):
        pltpu.make_async_copy(rows[r], accum.at[didx[slot]], sem_s[r]).wait()

    start_idx(0, 0)
    start_idx(1, 1)
    wait_idx(0)
    start_gather(0, 0)
    start_idx(2, 2)
    wait_idx(1)
    start_gather(1, 1)
    start_idx(3, 3)
    finish_chunk(0, 0)

    def body(t, _):
        for b in range(4):
            i = t * 4 + 2 + b
            sl = (2 + b) % 4
            r = b % 2
            wait_idx(sl)
            drain_scatter(b, r)
            start_gather(sl, r)
            start_idx(b, i + 2)
            finish_chunk((1 + b) % 4, 1 - r)
        return 0
    lax.fori_loop(0, (CPW - 4) // 4, body, 0)

    wait_idx(2)
    drain_scatter(0, 0)
    start_gather(2, 0)
    finish_chunk(1, 1)
    wait_idx(3)
    drain_scatter(1, 1)
    start_gather(3, 1)
    finish_chunk(2, 0)
    finish_chunk(3, 1)
    drain_scatter(2, 0)
    drain_scatter(3, 1)
    plsc.subcore_barrier()
    for t in range(RPT):
        rr = (s * RPT + t) * CH
        pltpu.sync_copy(accum.at[pl.ds(rr, CH)], out_hbm.at[c, pl.ds(rr, CH)])

  if with_ex:
    @kern
    def _edge_sc(hp_hbm, src_hbm, dst_hbm, ex_hbm, out_hbm, *rest):
      _body(hp_hbm, src_hbm, dst_hbm, ex_hbm, out_hbm, *rest)
  else:
    @kern
    def _edge_sc(hp_hbm, src_hbm, dst_hbm, out_hbm, *rest):
      _body(hp_hbm, src_hbm, dst_hbm, None, out_hbm, *rest)

  return _edge_sc


SPT = NR * HEADS // (NS * CH)


@functools.cache
def _att_kernel():
  kern = functools.partial(
    pl.kernel,
    out_type=(jax.ShapeDtypeStruct((HEADS, NE_PAD), jnp.float32),
              jax.ShapeDtypeStruct((NC, NR * HEADS), jnp.float32)),
    mesh=_mesh(),
    compiler_params=pltpu.CompilerParams(needs_layout_passes=False),
    scratch_types=[
        pltpu.VMEM((NR * HEADS,), jnp.float32),
        pltpu.VMEM((NR * HEADS,), jnp.float32),
        pltpu.VMEM((LANES,), jnp.float32),
        pltpu.VMEM((CH,), jnp.int32),
        pltpu.VMEM((CH,), jnp.int32),
        pltpu.VMEM((HEADS, CH), jnp.float32),
        pltpu.VMEM((HEADS, CH), jnp.int32),
        pltpu.VMEM((CH,), jnp.float32),
        pltpu.VMEM_SHARED((NR * HEADS,), jnp.float32),
    ],
  )

  @kern
  def _att_sc(as_hbm, ad_hbm, m_hbm, src_hbm, dst_hbm, ex_hbm, s_hbm,
              as_v, ad_v, m_v, sidx, didx, exb, sxb, zbuf, sacc):
    c = lax.axis_index("c")
    s = lax.axis_index("s")
    w = c * NS + s
    pltpu.sync_copy(as_hbm, as_v)
    pltpu.sync_copy(ad_hbm, ad_v)
    pltpu.sync_copy(m_hbm, m_v)
    mvec = m_v[...]
    for j in range(CH // LANES):
        zbuf[pl.ds(j * LANES, LANES)] = jnp.zeros((LANES,), jnp.float32)
    for t in range(SPT):
        pltpu.sync_copy(zbuf, sacc.at[pl.ds((s * SPT + t) * CH, CH)])
    plsc.subcore_barrier()

    def body(i, _):
        base = w * EPW + i * CH
        pltpu.sync_copy(src_hbm.at[pl.ds(base, CH)], sidx)
        pltpu.sync_copy(dst_hbm.at[pl.ds(base, CH)], didx)
        for g in range(CH // LANES):
            sv = sidx[pl.ds(g * LANES, LANES)] * HEADS
            dv = didx[pl.ds(g * LANES, LANES)] * HEADS
            for h in range(HEADS):
                av = plsc.load_gather(as_v, [sv + h])
                bv = plsc.load_gather(ad_v, [dv + h])
                z = av + bv
                e = jnp.where(z >= 0, z, z * 0.2) - mvec[h]
                exb[h, pl.ds(g * LANES, LANES)] = jnp.exp(e)
                sxb[h, pl.ds(g * LANES, LANES)] = dv + h
        for h in range(HEADS):
            pltpu.sync_copy(exb.at[h], ex_hbm.at[h, pl.ds(base, CH)])
            pltpu.sync_copy(exb.at[h], sacc.at[sxb.at[h]], add=True)
        return 0
    lax.fori_loop(0, CPW, body, 0)
    plsc.subcore_barrier()
    for t in range(SPT):
        r = (s * SPT + t) * CH
        pltpu.sync_copy(sacc.at[pl.ds(r, CH)], s_hbm.at[c, pl.ds(r, CH)])

  return _att_sc


def _gat_sc(attn, srcw, dstw, Wl, asl, adl, bl):
    Wcat = jnp.moveaxis(Wl, 0, 1).reshape(H, H)
    h = attn @ Wcat
    hh = h.reshape(N, HEADS, HD)
    asn = (hh * asl[None]).sum(-1)
    adn = (hh * adl[None]).sum(-1)
    M = jnp.max(asn, axis=0) + jnp.max(adn, axis=0)
    M = jnp.where(M >= 0, M, 0.2 * M)
    Mp = jnp.zeros((LANES,), jnp.float32).at[:HEADS].set(M)
    asp = jnp.zeros((NR, HEADS), jnp.float32).at[:N].set(asn).reshape(-1)
    adp = jnp.zeros((NR, HEADS), jnp.float32).at[:N].set(adn).reshape(-1)
    ex, s2 = _att_kernel()(asp, adp, Mp, srcw, dstw)
    sn = (s2[0] + s2[1]).reshape(NR, HEADS)[:N]
    hp = jnp.zeros((NR, H), jnp.float32).at[:N].set(h)
    agg2 = _edge_kernel(True)(hp, srcw, dstw, ex)
    agg = (agg2[0] + agg2[1])[:N].reshape(N, HEADS, HD)
    out = agg / (sn[:, :, None] + 1e-16) + bl[None]
    return out.reshape(N, H)


def _encoder_body(nf_ref, w1_ref, b1_ref, w2_ref, b2_ref, g_ref, bb_ref, o_ref):
    x = jnp.maximum(jnp.dot(nf_ref[...], w1_ref[...],
                            preferred_element_type=jnp.float32) + b1_ref[...], 0.0)
    x = jnp.dot(x, w2_ref[...], preferred_element_type=jnp.float32) + b2_ref[...]
    m = x.mean(-1, keepdims=True)
    v = ((x - m) ** 2).mean(-1, keepdims=True)
    o_ref[...] = (x - m) * lax.rsqrt(v + 1e-5) * g_ref[...] + bb_ref[...]


def _encoder(node_features, p):
    return pl.pallas_call(
        _encoder_body,
        grid=(N // NB,),
        in_specs=[
            pl.BlockSpec((NB, DF), lambda i: (i, 0)),
            pl.BlockSpec((DF, H), lambda i: (0, 0)),
            pl.BlockSpec((H,), lambda i: (0,)),
            pl.BlockSpec((H, H), lambda i: (0, 0)),
            pl.BlockSpec((H,), lambda i: (0,)),
            pl.BlockSpec((H,), lambda i: (0,)),
            pl.BlockSpec((H,), lambda i: (0,)),
        ],
        out_specs=pl.BlockSpec((NB, H), lambda i: (i, 0)),
        out_shape=jax.ShapeDtypeStruct((N, H), jnp.float32),
    )(node_features, p['enc_W1'], p['enc_b1'], p['enc_W2'], p['enc_b2'],
      p['enc_ln_g'], p['enc_ln_b'])


def _gat_jnp(x, src, dst, Wl, asl, adl, bl, n):
    heads = []
    for hh in range(HEADS):
        h = x @ Wl[hh]
        asrc = (h * asl[hh]).sum(-1)
        adst = (h * adl[hh]).sum(-1)
        e = jax.nn.leaky_relu(asrc[src] + adst[dst], 0.2)
        m = jax.ops.segment_max(e, dst, num_segments=n)
        ex = jnp.exp(e - m[dst])
        s = jax.ops.segment_sum(ex, dst, num_segments=n)
        alpha = ex / (s[dst] + 1e-16)
        heads.append(jax.ops.segment_sum(alpha[:, None] * h[src], dst, num_segments=n) + bl[hh])
    return jnp.concatenate(heads, axis=-1)


def kernel(node_features, edge_index, params):
    p = params
    n = N
    loop = jnp.arange(n, dtype=edge_index.dtype)
    src = jnp.concatenate([edge_index[0], loop])
    dst = jnp.concatenate([edge_index[1], loop])
    pad = jnp.full((NE_PAD - NE_TOT,), DUMMY, dtype=edge_index.dtype)
    srcw = jnp.concatenate([src, pad])
    dstw = jnp.concatenate([dst, pad])

    deg2 = _deg_kernel()(dstw)
    deg = (deg2[0] + deg2[1])[:n]
    dinv = jnp.where(deg > 0, 1.0 / jnp.sqrt(deg), 0.0)

    x = _encoder(node_features, p)

    attn = x
    for i in range(L):
        out = _gat_sc(attn, srcw, dstw, p['gat_W'][i], p['gat_asrc'][i],
                      p['gat_adst'][i], p['gat_b'][i])
        out = out @ p['proj_W'][i] + p['proj_b'][i]
        m = out.mean(-1, keepdims=True)
        v = ((out - m) ** 2).mean(-1, keepdims=True)
        out = (out - m) / jnp.sqrt(v + 1e-5) * p['ln_g'][i] + p['ln_b'][i]
        attn = attn + out

    trad = x
    for i in range(L):
        hp = jnp.zeros((NR, H), jnp.float32).at[:n].set(dinv[:, None] * (trad @ p['gcn_W'][i]))
        agg2 = _edge_kernel(False)(hp, srcw, dstw)
        agg = dinv[:, None] * (agg2[0] + agg2[1])[:n] + p['gcn_b'][i]
        trad = trad + jax.nn.relu(agg)

    combined = attn + trad
    g = jnp.concatenate([combined.mean(axis=0), combined.max(axis=0)])

    def mlp3(v, W1, b1, W2, b2, W3, b3):
        h1 = jax.nn.relu(v @ W1 + b1)
        h2 = jax.nn.relu(h1 @ W2 + b2)
        return h2 @ W3 + b3

    dec = jax.nn.sigmoid(mlp3(g, p['dec_W1'], p['dec_b1'], p['dec_W2'], p['dec_b2'], p['dec_W3'], p['dec_b3']))
    val = mlp3(g, p['val_W1'], p['val_b1'], p['val_W2'], p['val_b2'], p['val_W3'], p['val_b3'])
    temp = jax.nn.relu(g @ p['tmp_W1'] + p['tmp_b1']) @ p['tmp_W2'] + p['tmp_b2']
    safe = jax.nn.sigmoid(jax.nn.relu(g @ p['safe_W1'] + p['safe_b1']) @ p['safe_W2'] + p['safe_b2'])
    return dec, val, temp, safe


# speedup vs baseline: 1.8324x; 1.8319x over previous
import functools

import jax
import jax.numpy as jnp
from jax import lax
from jax.experimental import pallas as pl
from jax.experimental.pallas import tpu as pltpu
from jax.experimental.pallas import tpu_sc as plsc

N = 10000
E = 320000
DF = 128
H = 128
L = 4
HEADS = 4
HD = H // HEADS

NC = 2
NS = 16
LANES = 16
W = NC * NS

NE_TOT = E + N
CH = 128
CPW = -(-NE_TOT // (W * CH))
EPW = CPW * CH
NE_PAD = W * EPW
DUMMY = N
NR = 10240
RPT = NR // (NS * CH)

NB = 400

_mesh_cache = []


def _mesh():
    if not _mesh_cache:
        _mesh_cache.append(plsc.VectorSubcoreMesh(
            core_axis_name="c", subcore_axis_name="s",
            num_cores=NC, num_subcores=NS))
    return _mesh_cache[0]


def _zero_vmem_rows(rows):
    def zrow(i, _):
        for j in range(H // LANES):
            rows[i, pl.ds(j * LANES, LANES)] = jnp.zeros((LANES,), jnp.float32)
        return 0
    lax.fori_loop(0, CH, zrow, 0)


@functools.cache
def _deg_kernel():
  kern = functools.partial(
    pl.kernel,
    out_type=jax.ShapeDtypeStruct((NC, NR), jnp.float32),
    mesh=_mesh(),
    scratch_types=[
        pltpu.VMEM((CH,), jnp.int32),
        pltpu.VMEM((CH,), jnp.float32),
        pltpu.VMEM((CH,), jnp.float32),
        pltpu.VMEM_SHARED((NR,), jnp.float32),
    ],
  )

  @kern
  def _deg_sc(dst_hbm, out_hbm, didx, ones_v, zero_v, dacc):
    c = lax.axis_index("c")
    s = lax.axis_index("s")
    w = c * NS + s
    for j in range(CH // LANES):
        ones_v[pl.ds(j * LANES, LANES)] = jnp.ones((LANES,), jnp.float32)
        zero_v[pl.ds(j * LANES, LANES)] = jnp.zeros((LANES,), jnp.float32)
    for t in range(NR // (NS * CH)):
        pltpu.sync_copy(zero_v, dacc.at[pl.ds((s * RPT + t) * CH, CH)])
    plsc.subcore_barrier()

    def body(i, _):
        base = w * EPW + i * CH
        pltpu.sync_copy(dst_hbm.at[pl.ds(base, CH)], didx)
        pltpu.sync_copy(ones_v, dacc.at[didx], add=True)
        return 0
    lax.fori_loop(0, CPW, body, 0)
    plsc.subcore_barrier()
    for t in range(RPT):
        r = (s * RPT + t) * CH
        pltpu.sync_copy(dacc.at[pl.ds(r, CH)], out_hbm.at[c, pl.ds(r, CH)])

  return _deg_sc


@functools.cache
def _edge_kernel(with_ex):
  scratch = [
      pltpu.VMEM((CH,), jnp.int32),
      pltpu.VMEM((CH,), jnp.int32),
      pltpu.VMEM((CH, H), jnp.float32),
      pltpu.VMEM((HEADS, CH), jnp.float32),
      pltpu.VMEM_SHARED((NR, H), jnp.float32),
      pltpu.SemaphoreType.DMA,
  ]
  kern = functools.partial(
    pl.kernel,
    out_type=jax.ShapeDtypeStruct((NC, NR, H), jnp.float32),
    mesh=_mesh(),
    scratch_types=scratch,
  )

  def _body(hp_hbm, src_hbm, dst_hbm, ex_hbm, out_hbm,
            sidx, didx, rows, exb, accum, sem):
    c = lax.axis_index("c")
    s = lax.axis_index("s")
    w = c * NS + s
    _zero_vmem_rows(rows)
    for t in range(RPT):
        pltpu.sync_copy(rows, accum.at[pl.ds((s * RPT + t) * CH, CH)])
    plsc.subcore_barrier()

    def do_mul():
        if not with_ex:
            return

        def mul(g, _):
            exv = [exb[h, pl.ds(g * LANES, LANES)] for h in range(HEADS)]
            for el in range(LANES):
                e2 = g * LANES + el
                for h in range(HEADS):
                    x = exv[h][el]
                    for k2 in range(HD // LANES):
                        off = h * HD + k2 * LANES
                        rows[e2, pl.ds(off, LANES)] = rows[e2, pl.ds(off, LANES)] * x
            return 0
        lax.fori_loop(0, CH // LANES, mul, 0)

    def body(i, _):
        base = w * EPW + i * CH
        pltpu.sync_copy(src_hbm.at[pl.ds(base, CH)], sidx)
        pltpu.sync_copy(dst_hbm.at[pl.ds(base, CH)], didx)
        if with_ex:
            for h in range(HEADS):
                pltpu.sync_copy(ex_hbm.at[h, pl.ds(base, CH)], exb.at[h])
        pltpu.async_copy(hp_hbm.at[sidx], rows, sem).wait()
        do_mul()
        pltpu.sync_copy(rows, accum.at[didx], add=True)
        return 0
    lax.fori_loop(0, CPW, body, 0)
    plsc.subcore_barrier()
    for t in range(RPT):
        rr = (s * RPT + t) * CH
        pltpu.sync_copy(accum.at[pl.ds(rr, CH)], out_hbm.at[c, pl.ds(rr, CH)])

  if with_ex:
    @kern
    def _edge_sc(hp_hbm, src_hbm, dst_hbm, ex_hbm, out_hbm, *rest):
      _body(hp_hbm, src_hbm, dst_hbm, ex_hbm, out_hbm, *rest)
  else:
    @kern
    def _edge_sc(hp_hbm, src_hbm, dst_hbm, out_hbm, *rest):
      _body(hp_hbm, src_hbm, dst_hbm, None, out_hbm, *rest)

  return _edge_sc


SPT = NR * HEADS // (NS * CH)


@functools.cache
def _att_kernel():
  kern = functools.partial(
    pl.kernel,
    out_type=(jax.ShapeDtypeStruct((HEADS, NE_PAD), jnp.float32),
              jax.ShapeDtypeStruct((NC, NR * HEADS), jnp.float32)),
    mesh=_mesh(),
    compiler_params=pltpu.CompilerParams(needs_layout_passes=False),
    scratch_types=[
        pltpu.VMEM((NR * HEADS,), jnp.float32),
        pltpu.VMEM((NR * HEADS,), jnp.float32),
        pltpu.VMEM((LANES,), jnp.float32),
        pltpu.VMEM((CH,), jnp.int32),
        pltpu.VMEM((CH,), jnp.int32),
        pltpu.VMEM((HEADS, CH), jnp.float32),
        pltpu.VMEM((HEADS, CH), jnp.int32),
        pltpu.VMEM((CH,), jnp.float32),
        pltpu.VMEM_SHARED((NR * HEADS,), jnp.float32),
    ],
  )

  @kern
  def _att_sc(as_hbm, ad_hbm, m_hbm, src_hbm, dst_hbm, ex_hbm, s_hbm,
              as_v, ad_v, m_v, sidx, didx, exb, sxb, zbuf, sacc):
    c = lax.axis_index("c")
    s = lax.axis_index("s")
    w = c * NS + s
    pltpu.sync_copy(as_hbm, as_v)
    pltpu.sync_copy(ad_hbm, ad_v)
    pltpu.sync_copy(m_hbm, m_v)
    mvec = m_v[...]
    for j in range(CH // LANES):
        zbuf[pl.ds(j * LANES, LANES)] = jnp.zeros((LANES,), jnp.float32)
    for t in range(SPT):
        pltpu.sync_copy(zbuf, sacc.at[pl.ds((s * SPT + t) * CH, CH)])
    plsc.subcore_barrier()

    def body(i, _):
        base = w * EPW + i * CH
        pltpu.sync_copy(src_hbm.at[pl.ds(base, CH)], sidx)
        pltpu.sync_copy(dst_hbm.at[pl.ds(base, CH)], didx)
        for g in range(CH // LANES):
            sv = sidx[pl.ds(g * LANES, LANES)] * HEADS
            dv = didx[pl.ds(g * LANES, LANES)] * HEADS
            for h in range(HEADS):
                av = plsc.load_gather(as_v, [sv + h])
                bv = plsc.load_gather(ad_v, [dv + h])
                z = av + bv
                e = jnp.where(z >= 0, z, z * 0.2) - mvec[h]
                exb[h, pl.ds(g * LANES, LANES)] = jnp.exp(e)
                sxb[h, pl.ds(g * LANES, LANES)] = dv + h
        for h in range(HEADS):
            pltpu.sync_copy(exb.at[h], ex_hbm.at[h, pl.ds(base, CH)])
            pltpu.sync_copy(exb.at[h], sacc.at[sxb.at[h]], add=True)
        return 0
    lax.fori_loop(0, CPW, body, 0)
    plsc.subcore_barrier()
    for t in range(SPT):
        r = (s * SPT + t) * CH
        pltpu.sync_copy(sacc.at[pl.ds(r, CH)], s_hbm.at[c, pl.ds(r, CH)])

  return _att_sc


NBT = 512
PB = 2000


def _enc_tc(nfp, p):
    def body(nf_ref, w1_ref, b1_ref, w2_ref, b2_ref, g_ref, bb_ref, o_ref):
        x = jnp.maximum(jnp.dot(nf_ref[...], w1_ref[...],
                                preferred_element_type=jnp.float32) + b1_ref[...], 0.0)
        x = jnp.dot(x, w2_ref[...], preferred_element_type=jnp.float32) + b2_ref[...]
        m = x.mean(-1, keepdims=True)
        v = ((x - m) ** 2).mean(-1, keepdims=True)
        o_ref[...] = (x - m) * lax.rsqrt(v + 1e-5) * g_ref[...] + bb_ref[...]

    return pl.pallas_call(
        body, grid=(NR // NBT,),
        in_specs=[
            pl.BlockSpec((NBT, DF), lambda i: (i, 0)),
            pl.BlockSpec((DF, H), lambda i: (0, 0)),
            pl.BlockSpec((H,), lambda i: (0,)),
            pl.BlockSpec((H, H), lambda i: (0, 0)),
            pl.BlockSpec((H,), lambda i: (0,)),
            pl.BlockSpec((H,), lambda i: (0,)),
            pl.BlockSpec((H,), lambda i: (0,)),
        ],
        out_specs=pl.BlockSpec((NBT, H), lambda i: (i, 0)),
        out_shape=jax.ShapeDtypeStruct((NR, H), jnp.float32),
    )(nfp, p['enc_W1'], p['enc_b1'], p['enc_W2'], p['enc_b2'],
      p['enc_ln_g'], p['enc_ln_b'])


def _gat_pre_tc(attn, Wcat, asl, adl):
    def body(a_ref, w_ref, as_ref, ad_ref, h_ref, asn_ref, adn_ref):
        hblk = jnp.dot(a_ref[...], w_ref[...], preferred_element_type=jnp.float32)
        h_ref[...] = hblk
        av = as_ref[...]
        dv = ad_ref[...]
        ca, cd = [], []
        for h in range(HEADS):
            sl = hblk[:, h * HD:(h + 1) * HD]
            ca.append(jnp.sum(sl * av[h][None, :], axis=1, keepdims=True))
            cd.append(jnp.sum(sl * dv[h][None, :], axis=1, keepdims=True))
        asn_ref[...] = jnp.concatenate(ca, axis=1)
        adn_ref[...] = jnp.concatenate(cd, axis=1)

    return pl.pallas_call(
        body, grid=(NR // NBT,),
        in_specs=[
            pl.BlockSpec((NBT, H), lambda i: (i, 0)),
            pl.BlockSpec((H, H), lambda i: (0, 0)),
            pl.BlockSpec((HEADS, HD), lambda i: (0, 0)),
            pl.BlockSpec((HEADS, HD), lambda i: (0, 0)),
        ],
        out_specs=[
            pl.BlockSpec((NBT, H), lambda i: (i, 0)),
            pl.BlockSpec((NBT, HEADS), lambda i: (i, 0)),
            pl.BlockSpec((NBT, HEADS), lambda i: (i, 0)),
        ],
        out_shape=[
            jax.ShapeDtypeStruct((NR, H), jnp.float32),
            jax.ShapeDtypeStruct((NR, HEADS), jnp.float32),
            jax.ShapeDtypeStruct((NR, HEADS), jnp.float32),
        ],
    )(attn, Wcat, asl, adl)


def _gat_post_tc(agg2, s2r, attn, bcat, projW, projb, lng, lnb):
    def body(ag_ref, s_ref, a_ref, b_ref, pw_ref, pb_ref, g_ref, lb_ref, o_ref):
        agg = ag_ref[0] + ag_ref[1]
        sv = s_ref[0] + s_ref[1]
        parts = [agg[:, h * HD:(h + 1) * HD] / (sv[:, h:h + 1] + 1e-16)
                 for h in range(HEADS)]
        o = jnp.concatenate(parts, axis=1) + b_ref[...][None, :]
        y = jnp.dot(o, pw_ref[...], preferred_element_type=jnp.float32) + pb_ref[...]
        m = y.mean(-1, keepdims=True)
        v = ((y - m) ** 2).mean(-1, keepdims=True)
        y = (y - m) * lax.rsqrt(v + 1e-5) * g_ref[...] + lb_ref[...]
        o_ref[...] = a_ref[...] + y

    return pl.pallas_call(
        body, grid=(NR // NBT,),
        in_specs=[
            pl.BlockSpec((NC, NBT, H), lambda i: (0, i, 0)),
            pl.BlockSpec((NC, NBT, HEADS), lambda i: (0, i, 0)),
            pl.BlockSpec((NBT, H), lambda i: (i, 0)),
            pl.BlockSpec((H,), lambda i: (0,)),
            pl.BlockSpec((H, H), lambda i: (0, 0)),
            pl.BlockSpec((H,), lambda i: (0,)),
            pl.BlockSpec((H,), lambda i: (0,)),
            pl.BlockSpec((H,), lambda i: (0,)),
        ],
        out_specs=pl.BlockSpec((NBT, H), lambda i: (i, 0)),
        out_shape=jax.ShapeDtypeStruct((NR, H), jnp.float32),
    )(agg2, s2r, attn, bcat, projW, projb, lng, lnb)


def _gcn_pre_tc(xp, W0, deg2):
    def body(x_ref, w_ref, d_ref, hp_ref):
        deg = d_ref[0] + d_ref[1]
        dinv = jnp.where(deg > 0, lax.rsqrt(deg), 0.0)
        hp_ref[...] = dinv[:, None] * jnp.dot(x_ref[...], w_ref[...],
                                              preferred_element_type=jnp.float32)

    return pl.pallas_call(
        body, grid=(NR // NBT,),
        in_specs=[
            pl.BlockSpec((NBT, H), lambda i: (i, 0)),
            pl.BlockSpec((H, H), lambda i: (0, 0)),
            pl.BlockSpec((NC, NBT), lambda i: (0, i)),
        ],
        out_specs=pl.BlockSpec((NBT, H), lambda i: (i, 0)),
        out_shape=jax.ShapeDtypeStruct((NR, H), jnp.float32),
    )(xp, W0, deg2)


def _gcn_step_tc(agg2, deg2, trad, bvec, Wnext):
    def body(ag_ref, d_ref, t_ref, b_ref, wn_ref, tn_ref, hp_ref):
        deg = d_ref[0] + d_ref[1]
        dinv = jnp.where(deg > 0, lax.rsqrt(deg), 0.0)
        agg = dinv[:, None] * (ag_ref[0] + ag_ref[1]) + b_ref[...][None, :]
        tn = t_ref[...] + jnp.maximum(agg, 0.0)
        tn_ref[...] = tn
        hp_ref[...] = dinv[:, None] * jnp.dot(tn, wn_ref[...],
                                              preferred_element_type=jnp.float32)

    return pl.pallas_call(
        body, grid=(NR // NBT,),
        in_specs=[
            pl.BlockSpec((NC, NBT, H), lambda i: (0, i, 0)),
            pl.BlockSpec((NC, NBT), lambda i: (0, i)),
            pl.BlockSpec((NBT, H), lambda i: (i, 0)),
            pl.BlockSpec((H,), lambda i: (0,)),
            pl.BlockSpec((H, H), lambda i: (0, 0)),
        ],
        out_specs=[
            pl.BlockSpec((NBT, H), lambda i: (i, 0)),
            pl.BlockSpec((NBT, H), lambda i: (i, 0)),
        ],
        out_shape=[
            jax.ShapeDtypeStruct((NR, H), jnp.float32),
            jax.ShapeDtypeStruct((NR, H), jnp.float32),
        ],
    )(agg2, deg2, trad, bvec, Wnext)


def _pool_tc(attn, trad):
    def body(a_ref, t_ref, o_ref):
        cb = a_ref[...] + t_ref[...]

        @pl.when(pl.program_id(0) == 0)
        def _():
            o_ref[...] = jnp.full((8, H), -jnp.inf, jnp.float32)
            o_ref[0:1, :] = jnp.zeros((1, H), jnp.float32)

        o_ref[0:1, :] = o_ref[0:1, :] + jnp.sum(cb, axis=0, keepdims=True)
        o_ref[1:2, :] = jnp.maximum(o_ref[1:2, :], jnp.max(cb, axis=0, keepdims=True))

    return pl.pallas_call(
        body, grid=(N // PB,),
        in_specs=[
            pl.BlockSpec((PB, H), lambda i: (i, 0)),
            pl.BlockSpec((PB, H), lambda i: (i, 0)),
        ],
        out_specs=pl.BlockSpec((8, H), lambda i: (0, 0)),
        out_shape=jax.ShapeDtypeStruct((8, H), jnp.float32),
    )(attn, trad)


def _heads_tc(pool, p):
    def body(pool_ref, dw1, db1, dw2, db2, dw3, db3,
             vw1, vb1, vw2, vb2, vw3, vb3,
             tw1, tb1, tw2, tb2, sw1, sb1, sw2, sb2,
             dec_ref, val_ref, tmp_ref, safe_ref):
        mean = pool_ref[0:1, :] * (1.0 / N)
        mx = pool_ref[1:2, :]
        g = jnp.concatenate([mean, mx], axis=1)

        def mm(a, w_ref, b_ref):
            return jnp.dot(a, w_ref[...], preferred_element_type=jnp.float32) + b_ref[...]

        h1 = jnp.maximum(mm(g, dw1, db1), 0.0)
        h2 = jnp.maximum(mm(h1, dw2, db2), 0.0)
        dec_ref[...] = jax.nn.sigmoid(mm(h2, dw3, db3))
        h1 = jnp.maximum(mm(g, vw1, vb1), 0.0)
        h2 = jnp.maximum(mm(h1, vw2, vb2), 0.0)
        val_ref[...] = mm(h2, vw3, vb3)
        h1 = jnp.maximum(mm(g, tw1, tb1), 0.0)
        tmp_ref[...] = mm(h1, tw2, tb2)
        h1 = jnp.maximum(mm(g, sw1, sb1), 0.0)
        safe_ref[...] = jax.nn.sigmoid(mm(h1, sw2, sb2))

    names = ['dec_W1', 'dec_b1', 'dec_W2', 'dec_b2', 'dec_W3', 'dec_b3',
             'val_W1', 'val_b1', 'val_W2', 'val_b2', 'val_W3', 'val_b3',
             'tmp_W1', 'tmp_b1', 'tmp_W2', 'tmp_b2',
             'safe_W1', 'safe_b1', 'safe_W2', 'safe_b2']
    args = [pool] + [p[n] for n in names]
    return pl.pallas_call(
        body,
        out_shape=[
            jax.ShapeDtypeStruct((1, 4), jnp.float32),
            jax.ShapeDtypeStruct((1, 1), jnp.float32),
            jax.ShapeDtypeStruct((1, 3), jnp.float32),
            jax.ShapeDtypeStruct((1, 1), jnp.float32),
        ],
    )(*args)


def kernel(node_features, edge_index, params):
    p = params
    loop = jnp.arange(N, dtype=edge_index.dtype)
    pad = jnp.full((NE_PAD - NE_TOT,), DUMMY, dtype=edge_index.dtype)
    srcw = jnp.concatenate([edge_index[0], loop, pad])
    dstw = jnp.concatenate([edge_index[1], loop, pad])

    deg2 = _deg_kernel()(dstw)

    nfp = jnp.zeros((NR, DF), jnp.float32).at[:N].set(node_features)
    x = _enc_tc(nfp, p)

    attn = x
    for i in range(L):
        Wcat = jnp.moveaxis(p['gat_W'][i], 0, 1).reshape(H, H)
        h, asn, adn = _gat_pre_tc(attn, Wcat, p['gat_asrc'][i], p['gat_adst'][i])
        M = jnp.max(asn, axis=0) + jnp.max(adn, axis=0)
        M = jnp.where(M >= 0, M, 0.2 * M)
        Mp = jnp.zeros((LANES,), jnp.float32).at[:HEADS].set(M)
        ex, s2 = _att_kernel()(asn.reshape(-1), adn.reshape(-1), Mp, srcw, dstw)
        agg2 = _edge_kernel(True)(h, srcw, dstw, ex)
        attn = _gat_post_tc(agg2, s2.reshape(NC, NR, HEADS), attn,
                            p['gat_b'][i].reshape(H), p['proj_W'][i],
                            p['proj_b'][i], p['ln_g'][i], p['ln_b'][i])

    trad = x
    hp = _gcn_pre_tc(x, p['gcn_W'][0], deg2)
    for i in range(L):
        agg2 = _edge_kernel(False)(hp, srcw, dstw)
        Wnext = p['gcn_W'][(i + 1) % L]
        trad, hp = _gcn_step_tc(agg2, deg2, trad, p['gcn_b'][i], Wnext)

    pool = _pool_tc(attn, trad)
    dec, val, temp, safe = _heads_tc(pool, p)
    return (dec.reshape(4), val.reshape(1), temp.reshape(3), safe.reshape(1))


# speedup vs baseline: 2.2224x; 1.2128x over previous
import functools

import jax
import jax.numpy as jnp
from jax import lax
from jax.experimental import pallas as pl
from jax.experimental.pallas import tpu as pltpu
from jax.experimental.pallas import tpu_sc as plsc

N = 10000
E = 320000
DF = 128
H = 128
L = 4
HEADS = 4
HD = H // HEADS

NC = 2
NS = 16
LANES = 16
W = NC * NS

NE_TOT = E + N
CH = 128
CPW = -(-NE_TOT // (W * CH))
EPW = CPW * CH
NE_PAD = W * EPW
DUMMY = N
NR = 10240
RPT = NR // (NS * CH)

NB = 400

_mesh_cache = []


def _mesh():
    if not _mesh_cache:
        _mesh_cache.append(plsc.VectorSubcoreMesh(
            core_axis_name="c", subcore_axis_name="s",
            num_cores=NC, num_subcores=NS))
    return _mesh_cache[0]


def _zero_vmem_rows(rows):
    def zrow(i, _):
        for j in range(H // LANES):
            rows[i, pl.ds(j * LANES, LANES)] = jnp.zeros((LANES,), jnp.float32)
        return 0
    lax.fori_loop(0, CH, zrow, 0)


@functools.cache
def _deg_kernel():
  kern = functools.partial(
    pl.kernel,
    out_type=jax.ShapeDtypeStruct((NC, NR), jnp.float32),
    mesh=_mesh(),
    scratch_types=[
        pltpu.VMEM((CH,), jnp.int32),
        pltpu.VMEM((CH,), jnp.float32),
        pltpu.VMEM((CH,), jnp.float32),
        pltpu.VMEM_SHARED((NR,), jnp.float32),
    ],
  )

  @kern
  def _deg_sc(dst_hbm, out_hbm, didx, ones_v, zero_v, dacc):
    c = lax.axis_index("c")
    s = lax.axis_index("s")
    w = c * NS + s
    for j in range(CH // LANES):
        ones_v[pl.ds(j * LANES, LANES)] = jnp.ones((LANES,), jnp.float32)
        zero_v[pl.ds(j * LANES, LANES)] = jnp.zeros((LANES,), jnp.float32)
    for t in range(NR // (NS * CH)):
        pltpu.sync_copy(zero_v, dacc.at[pl.ds((s * RPT + t) * CH, CH)])
    plsc.subcore_barrier()

    def body(i, _):
        base = w * EPW + i * CH
        pltpu.sync_copy(dst_hbm.at[pl.ds(base, CH)], didx)
        pltpu.sync_copy(ones_v, dacc.at[didx], add=True)
        return 0
    lax.fori_loop(0, CPW, body, 0)
    plsc.subcore_barrier()
    for t in range(RPT):
        r = (s * RPT + t) * CH
        pltpu.sync_copy(dacc.at[pl.ds(r, CH)], out_hbm.at[c, pl.ds(r, CH)])

  return _deg_sc


@functools.cache
def _edge_kernel(with_ex):
  scratch = [
      pltpu.VMEM((CH,), jnp.int32),
      pltpu.VMEM((CH,), jnp.int32),
      pltpu.VMEM((CH, H), jnp.float32),
      pltpu.VMEM((HEADS, CH), jnp.float32),
      pltpu.VMEM_SHARED((NR, H), jnp.float32),
      pltpu.SemaphoreType.DMA,
  ]
  kern = functools.partial(
    pl.kernel,
    out_type=jax.ShapeDtypeStruct((NC, NR, H), jnp.float32),
    mesh=_mesh(),
    scratch_types=scratch,
  )

  def _body(hp_hbm, src_hbm, dst_hbm, ex_hbm, out_hbm,
            sidx, didx, rows, exb, accum, sem):
    c = lax.axis_index("c")
    s = lax.axis_index("s")
    w = c * NS + s
    _zero_vmem_rows(rows)
    for t in range(RPT):
        pltpu.sync_copy(rows, accum.at[pl.ds((s * RPT + t) * CH, CH)])
    plsc.subcore_barrier()

    def do_mul():
        if not with_ex:
            return

        def mul(g, _):
            exv = [exb[h, pl.ds(g * LANES, LANES)] for h in range(HEADS)]
            for el in range(LANES):
                e2 = g * LANES + el
                for h in range(HEADS):
                    x = exv[h][el]
                    for k2 in range(HD // LANES):
                        off = h * HD + k2 * LANES
                        rows[e2, pl.ds(off, LANES)] = rows[e2, pl.ds(off, LANES)] * x
            return 0
        lax.fori_loop(0, CH // LANES, mul, 0)

    def body(i, _):
        base = w * EPW + i * CH
        cps = [pltpu.async_copy(src_hbm.at[pl.ds(base, CH)], sidx, sem),
               pltpu.async_copy(dst_hbm.at[pl.ds(base, CH)], didx, sem)]
        if with_ex:
            for h in range(HEADS):
                cps.append(pltpu.async_copy(ex_hbm.at[h, pl.ds(base, CH)],
                                            exb.at[h], sem))
        for cp in cps:
            cp.wait()
        pltpu.async_copy(hp_hbm.at[sidx], rows, sem).wait()
        do_mul()
        pltpu.sync_copy(rows, accum.at[didx], add=True)
        return 0
    lax.fori_loop(0, CPW, body, 0)
    plsc.subcore_barrier()
    for t in range(RPT):
        rr = (s * RPT + t) * CH
        pltpu.sync_copy(accum.at[pl.ds(rr, CH)], out_hbm.at[c, pl.ds(rr, CH)])

  if with_ex:
    @kern
    def _edge_sc(hp_hbm, src_hbm, dst_hbm, ex_hbm, out_hbm, *rest):
      _body(hp_hbm, src_hbm, dst_hbm, ex_hbm, out_hbm, *rest)
  else:
    @kern
    def _edge_sc(hp_hbm, src_hbm, dst_hbm, out_hbm, *rest):
      _body(hp_hbm, src_hbm, dst_hbm, None, out_hbm, *rest)

  return _edge_sc


SPT = NR * HEADS // (NS * CH)


@functools.cache
def _att_kernel():
  kern = functools.partial(
    pl.kernel,
    out_type=(jax.ShapeDtypeStruct((HEADS, NE_PAD), jnp.float32),
              jax.ShapeDtypeStruct((NC, NR * HEADS), jnp.float32)),
    mesh=_mesh(),
    compiler_params=pltpu.CompilerParams(needs_layout_passes=False),
    scratch_types=[
        pltpu.VMEM((NR * HEADS,), jnp.float32),
        pltpu.VMEM((NR * HEADS,), jnp.float32),
        pltpu.VMEM((LANES,), jnp.float32),
        pltpu.VMEM((CH,), jnp.int32),
        pltpu.VMEM((CH,), jnp.int32),
        pltpu.VMEM((HEADS, CH), jnp.float32),
        pltpu.VMEM((HEADS, CH), jnp.int32),
        pltpu.VMEM((CH,), jnp.float32),
        pltpu.VMEM_SHARED((NR * HEADS,), jnp.float32),
    ],
  )

  @kern
  def _att_sc(as_hbm, ad_hbm, m_hbm, src_hbm, dst_hbm, ex_hbm, s_hbm,
              as_v, ad_v, m_v, sidx, didx, exb, sxb, zbuf, sacc):
    c = lax.axis_index("c")
    s = lax.axis_index("s")
    w = c * NS + s
    pltpu.sync_copy(as_hbm, as_v)
    pltpu.sync_copy(ad_hbm, ad_v)
    pltpu.sync_copy(m_hbm, m_v)
    mvec = m_v[...]
    for j in range(CH // LANES):
        zbuf[pl.ds(j * LANES, LANES)] = jnp.zeros((LANES,), jnp.float32)
    for t in range(SPT):
        pltpu.sync_copy(zbuf, sacc.at[pl.ds((s * SPT + t) * CH, CH)])
    plsc.subcore_barrier()

    def body(i, _):
        base = w * EPW + i * CH
        pltpu.sync_copy(src_hbm.at[pl.ds(base, CH)], sidx)
        pltpu.sync_copy(dst_hbm.at[pl.ds(base, CH)], didx)
        for g in range(CH // LANES):
            sv = sidx[pl.ds(g * LANES, LANES)] * HEADS
            dv = didx[pl.ds(g * LANES, LANES)] * HEADS
            for h in range(HEADS):
                av = plsc.load_gather(as_v, [sv + h])
                bv = plsc.load_gather(ad_v, [dv + h])
                z = av + bv
                e = jnp.where(z >= 0, z, z * 0.2) - mvec[h]
                exb[h, pl.ds(g * LANES, LANES)] = jnp.exp(e)
                sxb[h, pl.ds(g * LANES, LANES)] = dv + h
        for h in range(HEADS):
            pltpu.sync_copy(exb.at[h], ex_hbm.at[h, pl.ds(base, CH)])
            pltpu.sync_copy(exb.at[h], sacc.at[sxb.at[h]], add=True)
        return 0
    lax.fori_loop(0, CPW, body, 0)
    plsc.subcore_barrier()
    for t in range(SPT):
        r = (s * SPT + t) * CH
        pltpu.sync_copy(sacc.at[pl.ds(r, CH)], s_hbm.at[c, pl.ds(r, CH)])

  return _att_sc


NBT = 512
PB = 2000


def _enc_tc(nfp, p):
    def body(nf_ref, w1_ref, b1_ref, w2_ref, b2_ref, g_ref, bb_ref, o_ref):
        x = jnp.maximum(jnp.dot(nf_ref[...], w1_ref[...],
                                preferred_element_type=jnp.float32) + b1_ref[...], 0.0)
        x = jnp.dot(x, w2_ref[...], preferred_element_type=jnp.float32) + b2_ref[...]
        m = x.mean(-1, keepdims=True)
        v = ((x - m) ** 2).mean(-1, keepdims=True)
        o_ref[...] = (x - m) * lax.rsqrt(v + 1e-5) * g_ref[...] + bb_ref[...]

    return pl.pallas_call(
        body, grid=(NR // NBT,),
        in_specs=[
            pl.BlockSpec((NBT, DF), lambda i: (i, 0)),
            pl.BlockSpec((DF, H), lambda i: (0, 0)),
            pl.BlockSpec((H,), lambda i: (0,)),
            pl.BlockSpec((H, H), lambda i: (0, 0)),
            pl.BlockSpec((H,), lambda i: (0,)),
            pl.BlockSpec((H,), lambda i: (0,)),
            pl.BlockSpec((H,), lambda i: (0,)),
        ],
        out_specs=pl.BlockSpec((NBT, H), lambda i: (i, 0)),
        out_shape=jax.ShapeDtypeStruct((NR, H), jnp.float32),
    )(nfp, p['enc_W1'], p['enc_b1'], p['enc_W2'], p['enc_b2'],
      p['enc_ln_g'], p['enc_ln_b'])


def _gat_pre_tc(attn, Wcat, asl, adl):
    def body(a_ref, w_ref, as_ref, ad_ref, h_ref, asn_ref, adn_ref):
        hblk = jnp.dot(a_ref[...], w_ref[...], preferred_element_type=jnp.float32)
        h_ref[...] = hblk
        av = as_ref[...]
        dv = ad_ref[...]
        ca, cd = [], []
        for h in range(HEADS):
            sl = hblk[:, h * HD:(h + 1) * HD]
            ca.append(jnp.sum(sl * av[h][None, :], axis=1, keepdims=True))
            cd.append(jnp.sum(sl * dv[h][None, :], axis=1, keepdims=True))
        asn_ref[...] = jnp.concatenate(ca, axis=1)
        adn_ref[...] = jnp.concatenate(cd, axis=1)

    return pl.pallas_call(
        body, grid=(NR // NBT,),
        in_specs=[
            pl.BlockSpec((NBT, H), lambda i: (i, 0)),
            pl.BlockSpec((H, H), lambda i: (0, 0)),
            pl.BlockSpec((HEADS, HD), lambda i: (0, 0)),
            pl.BlockSpec((HEADS, HD), lambda i: (0, 0)),
        ],
        out_specs=[
            pl.BlockSpec((NBT, H), lambda i: (i, 0)),
            pl.BlockSpec((NBT, HEADS), lambda i: (i, 0)),
            pl.BlockSpec((NBT, HEADS), lambda i: (i, 0)),
        ],
        out_shape=[
            jax.ShapeDtypeStruct((NR, H), jnp.float32),
            jax.ShapeDtypeStruct((NR, HEADS), jnp.float32),
            jax.ShapeDtypeStruct((NR, HEADS), jnp.float32),
        ],
    )(attn, Wcat, asl, adl)


def _gat_post_tc(agg2, s2r, attn, bcat, projW, projb, lng, lnb):
    def body(ag_ref, s_ref, a_ref, b_ref, pw_ref, pb_ref, g_ref, lb_ref, o_ref):
        agg = ag_ref[0] + ag_ref[1]
        sv = s_ref[0] + s_ref[1]
        parts = [agg[:, h * HD:(h + 1) * HD] / (sv[:, h:h + 1] + 1e-16)
                 for h in range(HEADS)]
        o = jnp.concatenate(parts, axis=1) + b_ref[...][None, :]
        y = jnp.dot(o, pw_ref[...], preferred_element_type=jnp.float32) + pb_ref[...]
        m = y.mean(-1, keepdims=True)
        v = ((y - m) ** 2).mean(-1, keepdims=True)
        y = (y - m) * lax.rsqrt(v + 1e-5) * g_ref[...] + lb_ref[...]
        o_ref[...] = a_ref[...] + y

    return pl.pallas_call(
        body, grid=(NR // NBT,),
        in_specs=[
            pl.BlockSpec((NC, NBT, H), lambda i: (0, i, 0)),
            pl.BlockSpec((NC, NBT, HEADS), lambda i: (0, i, 0)),
            pl.BlockSpec((NBT, H), lambda i: (i, 0)),
            pl.BlockSpec((H,), lambda i: (0,)),
            pl.BlockSpec((H, H), lambda i: (0, 0)),
            pl.BlockSpec((H,), lambda i: (0,)),
            pl.BlockSpec((H,), lambda i: (0,)),
            pl.BlockSpec((H,), lambda i: (0,)),
        ],
        out_specs=pl.BlockSpec((NBT, H), lambda i: (i, 0)),
        out_shape=jax.ShapeDtypeStruct((NR, H), jnp.float32),
    )(agg2, s2r, attn, bcat, projW, projb, lng, lnb)


def _gcn_pre_tc(xp, W0, deg2):
    def body(x_ref, w_ref, d_ref, hp_ref):
        deg = d_ref[0] + d_ref[1]
        dinv = jnp.where(deg > 0, lax.rsqrt(deg), 0.0)
        hp_ref[...] = dinv[:, None] * jnp.dot(x_ref[...], w_ref[...],
                                              preferred_element_type=jnp.float32)

    return pl.pallas_call(
        body, grid=(NR // NBT,),
        in_specs=[
            pl.BlockSpec((NBT, H), lambda i: (i, 0)),
            pl.BlockSpec((H, H), lambda i: (0, 0)),
            pl.BlockSpec((NC, NBT), lambda i: (0, i)),
        ],
        out_specs=pl.BlockSpec((NBT, H), lambda i: (i, 0)),
        out_shape=jax.ShapeDtypeStruct((NR, H), jnp.float32),
    )(xp, W0, deg2)


def _gcn_step_tc(agg2, deg2, trad, bvec, Wnext):
    def body(ag_ref, d_ref, t_ref, b_ref, wn_ref, tn_ref, hp_ref):
        deg = d_ref[0] + d_ref[1]
        dinv = jnp.where(deg > 0, lax.rsqrt(deg), 0.0)
        agg = dinv[:, None] * (ag_ref[0] + ag_ref[1]) + b_ref[...][None, :]
        tn = t_ref[...] + jnp.maximum(agg, 0.0)
        tn_ref[...] = tn
        hp_ref[...] = dinv[:, None] * jnp.dot(tn, wn_ref[...],
                                              preferred_element_type=jnp.float32)

    return pl.pallas_call(
        body, grid=(NR // NBT,),
        in_specs=[
            pl.BlockSpec((NC, NBT, H), lambda i: (0, i, 0)),
            pl.BlockSpec((NC, NBT), lambda i: (0, i)),
            pl.BlockSpec((NBT, H), lambda i: (i, 0)),
            pl.BlockSpec((H,), lambda i: (0,)),
            pl.BlockSpec((H, H), lambda i: (0, 0)),
        ],
        out_specs=[
            pl.BlockSpec((NBT, H), lambda i: (i, 0)),
            pl.BlockSpec((NBT, H), lambda i: (i, 0)),
        ],
        out_shape=[
            jax.ShapeDtypeStruct((NR, H), jnp.float32),
            jax.ShapeDtypeStruct((NR, H), jnp.float32),
        ],
    )(agg2, deg2, trad, bvec, Wnext)


def _pool_tc(attn, trad):
    def body(a_ref, t_ref, o_ref):
        cb = a_ref[...] + t_ref[...]

        @pl.when(pl.program_id(0) == 0)
        def _():
            o_ref[...] = jnp.full((8, H), -jnp.inf, jnp.float32)
            o_ref[0:1, :] = jnp.zeros((1, H), jnp.float32)

        o_ref[0:1, :] = o_ref[0:1, :] + jnp.sum(cb, axis=0, keepdims=True)
        o_ref[1:2, :] = jnp.maximum(o_ref[1:2, :], jnp.max(cb, axis=0, keepdims=True))

    return pl.pallas_call(
        body, grid=(N // PB,),
        in_specs=[
            pl.BlockSpec((PB, H), lambda i: (i, 0)),
            pl.BlockSpec((PB, H), lambda i: (i, 0)),
        ],
        out_specs=pl.BlockSpec((8, H), lambda i: (0, 0)),
        out_shape=jax.ShapeDtypeStruct((8, H), jnp.float32),
    )(attn, trad)


def _heads_tc(pool, p):
    def body(pool_ref, dw1, db1, dw2, db2, dw3, db3,
             vw1, vb1, vw2, vb2, vw3, vb3,
             tw1, tb1, tw2, tb2, sw1, sb1, sw2, sb2,
             dec_ref, val_ref, tmp_ref, safe_ref):
        mean = pool_ref[0:1, :] * (1.0 / N)
        mx = pool_ref[1:2, :]
        g = jnp.concatenate([mean, mx], axis=1)

        def mm(a, w_ref, b_ref):
            return jnp.dot(a, w_ref[...], preferred_element_type=jnp.float32) + b_ref[...]

        h1 = jnp.maximum(mm(g, dw1, db1), 0.0)
        h2 = jnp.maximum(mm(h1, dw2, db2), 0.0)
        dec_ref[...] = jax.nn.sigmoid(mm(h2, dw3, db3))
        h1 = jnp.maximum(mm(g, vw1, vb1), 0.0)
        h2 = jnp.maximum(mm(h1, vw2, vb2), 0.0)
        val_ref[...] = mm(h2, vw3, vb3)
        h1 = jnp.maximum(mm(g, tw1, tb1), 0.0)
        tmp_ref[...] = mm(h1, tw2, tb2)
        h1 = jnp.maximum(mm(g, sw1, sb1), 0.0)
        safe_ref[...] = jax.nn.sigmoid(mm(h1, sw2, sb2))

    names = ['dec_W1', 'dec_b1', 'dec_W2', 'dec_b2', 'dec_W3', 'dec_b3',
             'val_W1', 'val_b1', 'val_W2', 'val_b2', 'val_W3', 'val_b3',
             'tmp_W1', 'tmp_b1', 'tmp_W2', 'tmp_b2',
             'safe_W1', 'safe_b1', 'safe_W2', 'safe_b2']
    args = [pool] + [p[n] for n in names]
    return pl.pallas_call(
        body,
        out_shape=[
            jax.ShapeDtypeStruct((1, 4), jnp.float32),
            jax.ShapeDtypeStruct((1, 1), jnp.float32),
            jax.ShapeDtypeStruct((1, 3), jnp.float32),
            jax.ShapeDtypeStruct((1, 1), jnp.float32),
        ],
    )(*args)


def kernel(node_features, edge_index, params):
    p = params
    loop = jnp.arange(N, dtype=edge_index.dtype)
    pad = jnp.full((NE_PAD - NE_TOT,), DUMMY, dtype=edge_index.dtype)
    srcw = jnp.concatenate([edge_index[0], loop, pad])
    dstw = jnp.concatenate([edge_index[1], loop, pad])

    deg2 = _deg_kernel()(dstw)

    nfp = jnp.zeros((NR, DF), jnp.float32).at[:N].set(node_features)
    x = _enc_tc(nfp, p)

    attn = x
    for i in range(L):
        Wcat = jnp.moveaxis(p['gat_W'][i], 0, 1).reshape(H, H)
        h, asn, adn = _gat_pre_tc(attn, Wcat, p['gat_asrc'][i], p['gat_adst'][i])
        M = jnp.max(asn, axis=0) + jnp.max(adn, axis=0)
        M = jnp.where(M >= 0, M, 0.2 * M)
        Mp = jnp.zeros((LANES,), jnp.float32).at[:HEADS].set(M)
        ex, s2 = _att_kernel()(asn.reshape(-1), adn.reshape(-1), Mp, srcw, dstw)
        agg2 = _edge_kernel(True)(h, srcw, dstw, ex)
        attn = _gat_post_tc(agg2, s2.reshape(NC, NR, HEADS), attn,
                            p['gat_b'][i].reshape(H), p['proj_W'][i],
                            p['proj_b'][i], p['ln_g'][i], p['ln_b'][i])

    trad = x
    hp = _gcn_pre_tc(x, p['gcn_W'][0], deg2)
    for i in range(L):
        agg2 = _edge_kernel(False)(hp, srcw, dstw)
        Wnext = p['gcn_W'][(i + 1) % L]
        trad, hp = _gcn_step_tc(agg2, deg2, trad, p['gcn_b'][i], Wnext)

    pool = _pool_tc(attn, trad)
    dec, val, temp, safe = _heads_tc(pool, p)
    return (dec.reshape(4), val.reshape(1), temp.reshape(3), safe.reshape(1))


# speedup vs baseline: 2.2275x; 1.0023x over previous
import functools

import jax
import jax.numpy as jnp
from jax import lax
from jax.experimental import pallas as pl
from jax.experimental.pallas import tpu as pltpu
from jax.experimental.pallas import tpu_sc as plsc

N = 10000
E = 320000
DF = 128
H = 128
L = 4
HEADS = 4
HD = H // HEADS

NC = 2
NS = 16
LANES = 16
W = NC * NS

NE_TOT = E + N
CH = 128
CPW = -(-NE_TOT // (W * CH))
EPW = CPW * CH
NE_PAD = W * EPW
DUMMY = N
NR = 10240
RPT = NR // (NS * CH)

NB = 400

_mesh_cache = []


def _mesh():
    if not _mesh_cache:
        _mesh_cache.append(plsc.VectorSubcoreMesh(
            core_axis_name="c", subcore_axis_name="s",
            num_cores=NC, num_subcores=NS))
    return _mesh_cache[0]


def _zero_vmem_rows(rows):
    def zrow(i, _):
        for j in range(H // LANES):
            rows[i, pl.ds(j * LANES, LANES)] = jnp.zeros((LANES,), jnp.float32)
        return 0
    lax.fori_loop(0, CH, zrow, 0)


@functools.cache
def _deg_kernel():
  kern = functools.partial(
    pl.kernel,
    out_type=jax.ShapeDtypeStruct((NC, NR), jnp.float32),
    mesh=_mesh(),
    scratch_types=[
        pltpu.VMEM((CH,), jnp.int32),
        pltpu.VMEM((CH,), jnp.float32),
        pltpu.VMEM((CH,), jnp.float32),
        pltpu.VMEM_SHARED((NR,), jnp.float32),
    ],
  )

  @kern
  def _deg_sc(dst_hbm, out_hbm, didx, ones_v, zero_v, dacc):
    c = lax.axis_index("c")
    s = lax.axis_index("s")
    w = c * NS + s
    for j in range(CH // LANES):
        ones_v[pl.ds(j * LANES, LANES)] = jnp.ones((LANES,), jnp.float32)
        zero_v[pl.ds(j * LANES, LANES)] = jnp.zeros((LANES,), jnp.float32)
    for t in range(NR // (NS * CH)):
        pltpu.sync_copy(zero_v, dacc.at[pl.ds((s * RPT + t) * CH, CH)])
    plsc.subcore_barrier()

    def body(i, _):
        base = w * EPW + i * CH
        pltpu.sync_copy(dst_hbm.at[pl.ds(base, CH)], didx)
        pltpu.sync_copy(ones_v, dacc.at[didx], add=True)
        return 0
    lax.fori_loop(0, CPW, body, 0)
    plsc.subcore_barrier()
    for t in range(RPT):
        r = (s * RPT + t) * CH
        pltpu.sync_copy(dacc.at[pl.ds(r, CH)], out_hbm.at[c, pl.ds(r, CH)])

  return _deg_sc


@functools.cache
def _edge_kernel(with_ex):
  scratch = [
      pltpu.VMEM((CH,), jnp.int32),
      pltpu.VMEM((CH,), jnp.int32),
      pltpu.VMEM((CH, H), jnp.float32),
      pltpu.VMEM((HEADS, CH), jnp.float32),
      pltpu.VMEM_SHARED((NR, H), jnp.float32),
      pltpu.SemaphoreType.DMA,
      pltpu.SemaphoreType.DMA,
  ]
  kern = functools.partial(
    pl.kernel,
    out_type=jax.ShapeDtypeStruct((NC, NR, H), jnp.float32),
    mesh=_mesh(),
    scratch_types=scratch,
  )

  def _body(hp_hbm, src_hbm, dst_hbm, ex_hbm, out_hbm,
            sidx, didx, rows, exb, accum, sem, sem2):
    c = lax.axis_index("c")
    s = lax.axis_index("s")
    w = c * NS + s
    _zero_vmem_rows(rows)
    for t in range(RPT):
        pltpu.sync_copy(rows, accum.at[pl.ds((s * RPT + t) * CH, CH)])
    plsc.subcore_barrier()

    def do_mul():
        if not with_ex:
            return

        def mul(g, _):
            exv = [exb[h, pl.ds(g * LANES, LANES)] for h in range(HEADS)]
            for el in range(LANES):
                e2 = g * LANES + el
                for h in range(HEADS):
                    x = exv[h][el]
                    for k2 in range(HD // LANES):
                        off = h * HD + k2 * LANES
                        rows[e2, pl.ds(off, LANES)] = rows[e2, pl.ds(off, LANES)] * x
            return 0
        lax.fori_loop(0, CH // LANES, mul, 0)

    def body(i, _):
        base = w * EPW + i * CH
        cps = [pltpu.async_copy(src_hbm.at[pl.ds(base, CH)], sidx, sem2),
               pltpu.async_copy(dst_hbm.at[pl.ds(base, CH)], didx, sem2)]
        if with_ex:
            for h in range(HEADS):
                cps.append(pltpu.async_copy(ex_hbm.at[h, pl.ds(base, CH)],
                                            exb.at[h], sem2))
        for cp in cps:
            cp.wait()
        pltpu.async_copy(hp_hbm.at[sidx], rows, sem).wait()
        do_mul()
        pltpu.sync_copy(rows, accum.at[didx], add=True)
        return 0
    lax.fori_loop(0, CPW, body, 0)
    plsc.subcore_barrier()
    for t in range(RPT):
        rr = (s * RPT + t) * CH
        pltpu.sync_copy(accum.at[pl.ds(rr, CH)], out_hbm.at[c, pl.ds(rr, CH)])

  if with_ex:
    @kern
    def _edge_sc(hp_hbm, src_hbm, dst_hbm, ex_hbm, out_hbm, *rest):
      _body(hp_hbm, src_hbm, dst_hbm, ex_hbm, out_hbm, *rest)
  else:
    @kern
    def _edge_sc(hp_hbm, src_hbm, dst_hbm, out_hbm, *rest):
      _body(hp_hbm, src_hbm, dst_hbm, None, out_hbm, *rest)

  return _edge_sc


SPT = NR * HEADS // (NS * CH)


@functools.cache
def _att_kernel():
  kern = functools.partial(
    pl.kernel,
    out_type=(jax.ShapeDtypeStruct((HEADS, NE_PAD), jnp.float32),
              jax.ShapeDtypeStruct((NC, NR * HEADS), jnp.float32)),
    mesh=_mesh(),
    compiler_params=pltpu.CompilerParams(needs_layout_passes=False),
    scratch_types=[
        pltpu.VMEM((NR * HEADS,), jnp.float32),
        pltpu.VMEM((NR * HEADS,), jnp.float32),
        pltpu.VMEM((LANES,), jnp.float32),
        pltpu.VMEM((CH,), jnp.int32),
        pltpu.VMEM((CH,), jnp.int32),
        pltpu.VMEM((HEADS, CH), jnp.float32),
        pltpu.VMEM((HEADS, CH), jnp.int32),
        pltpu.VMEM((CH,), jnp.float32),
        pltpu.VMEM_SHARED((NR * HEADS,), jnp.float32),
    ],
  )

  @kern
  def _att_sc(as_hbm, ad_hbm, m_hbm, src_hbm, dst_hbm, ex_hbm, s_hbm,
              as_v, ad_v, m_v, sidx, didx, exb, sxb, zbuf, sacc):
    c = lax.axis_index("c")
    s = lax.axis_index("s")
    w = c * NS + s
    pltpu.sync_copy(as_hbm, as_v)
    pltpu.sync_copy(ad_hbm, ad_v)
    pltpu.sync_copy(m_hbm, m_v)
    mvec = m_v[...]
    for j in range(CH // LANES):
        zbuf[pl.ds(j * LANES, LANES)] = jnp.zeros((LANES,), jnp.float32)
    for t in range(SPT):
        pltpu.sync_copy(zbuf, sacc.at[pl.ds((s * SPT + t) * CH, CH)])
    plsc.subcore_barrier()

    def body(i, _):
        base = w * EPW + i * CH
        pltpu.sync_copy(src_hbm.at[pl.ds(base, CH)], sidx)
        pltpu.sync_copy(dst_hbm.at[pl.ds(base, CH)], didx)
        for g in range(CH // LANES):
            sv = sidx[pl.ds(g * LANES, LANES)] * HEADS
            dv = didx[pl.ds(g * LANES, LANES)] * HEADS
            for h in range(HEADS):
                av = plsc.load_gather(as_v, [sv + h])
                bv = plsc.load_gather(ad_v, [dv + h])
                z = av + bv
                e = jnp.where(z >= 0, z, z * 0.2) - mvec[h]
                exb[h, pl.ds(g * LANES, LANES)] = jnp.exp(e)
                sxb[h, pl.ds(g * LANES, LANES)] = dv + h
        for h in range(HEADS):
            pltpu.sync_copy(exb.at[h], ex_hbm.at[h, pl.ds(base, CH)])
            pltpu.sync_copy(exb.at[h], sacc.at[sxb.at[h]], add=True)
        return 0
    lax.fori_loop(0, CPW, body, 0)
    plsc.subcore_barrier()
    for t in range(SPT):
        r = (s * SPT + t) * CH
        pltpu.sync_copy(sacc.at[pl.ds(r, CH)], s_hbm.at[c, pl.ds(r, CH)])

  return _att_sc


NBT = 512
PB = 2000


def _enc_tc(nfp, p):
    def body(nf_ref, w1_ref, b1_ref, w2_ref, b2_ref, g_ref, bb_ref, o_ref):
        x = jnp.maximum(jnp.dot(nf_ref[...], w1_ref[...],
                                preferred_element_type=jnp.float32) + b1_ref[...], 0.0)
        x = jnp.dot(x, w2_ref[...], preferred_element_type=jnp.float32) + b2_ref[...]
        m = x.mean(-1, keepdims=True)
        v = ((x - m) ** 2).mean(-1, keepdims=True)
        o_ref[...] = (x - m) * lax.rsqrt(v + 1e-5) * g_ref[...] + bb_ref[...]

    return pl.pallas_call(
        body, grid=(NR // NBT,),
        in_specs=[
            pl.BlockSpec((NBT, DF), lambda i: (i, 0)),
            pl.BlockSpec((DF, H), lambda i: (0, 0)),
            pl.BlockSpec((H,), lambda i: (0,)),
            pl.BlockSpec((H, H), lambda i: (0, 0)),
            pl.BlockSpec((H,), lambda i: (0,)),
            pl.BlockSpec((H,), lambda i: (0,)),
            pl.BlockSpec((H,), lambda i: (0,)),
        ],
        out_specs=pl.BlockSpec((NBT, H), lambda i: (i, 0)),
        out_shape=jax.ShapeDtypeStruct((NR, H), jnp.float32),
    )(nfp, p['enc_W1'], p['enc_b1'], p['enc_W2'], p['enc_b2'],
      p['enc_ln_g'], p['enc_ln_b'])


def _gat_pre_tc(attn, Wcat, asl, adl):
    def body(a_ref, w_ref, as_ref, ad_ref, h_ref, asn_ref, adn_ref):
        hblk = jnp.dot(a_ref[...], w_ref[...], preferred_element_type=jnp.float32)
        h_ref[...] = hblk
        av = as_ref[...]
        dv = ad_ref[...]
        ca, cd = [], []
        for h in range(HEADS):
            sl = hblk[:, h * HD:(h + 1) * HD]
            ca.append(jnp.sum(sl * av[h][None, :], axis=1, keepdims=True))
            cd.append(jnp.sum(sl * dv[h][None, :], axis=1, keepdims=True))
        asn_ref[...] = jnp.concatenate(ca, axis=1)
        adn_ref[...] = jnp.concatenate(cd, axis=1)

    return pl.pallas_call(
        body, grid=(NR // NBT,),
        in_specs=[
            pl.BlockSpec((NBT, H), lambda i: (i, 0)),
            pl.BlockSpec((H, H), lambda i: (0, 0)),
            pl.BlockSpec((HEADS, HD), lambda i: (0, 0)),
            pl.BlockSpec((HEADS, HD), lambda i: (0, 0)),
        ],
        out_specs=[
            pl.BlockSpec((NBT, H), lambda i: (i, 0)),
            pl.BlockSpec((NBT, HEADS), lambda i: (i, 0)),
            pl.BlockSpec((NBT, HEADS), lambda i: (i, 0)),
        ],
        out_shape=[
            jax.ShapeDtypeStruct((NR, H), jnp.float32),
            jax.ShapeDtypeStruct((NR, HEADS), jnp.float32),
            jax.ShapeDtypeStruct((NR, HEADS), jnp.float32),
        ],
    )(attn, Wcat, asl, adl)


def _gat_post_tc(agg2, s2r, attn, bcat, projW, projb, lng, lnb):
    def body(ag_ref, s_ref, a_ref, b_ref, pw_ref, pb_ref, g_ref, lb_ref, o_ref):
        agg = ag_ref[0] + ag_ref[1]
        sv = s_ref[0] + s_ref[1]
        parts = [agg[:, h * HD:(h + 1) * HD] / (sv[:, h:h + 1] + 1e-16)
                 for h in range(HEADS)]
        o = jnp.concatenate(parts, axis=1) + b_ref[...][None, :]
        y = jnp.dot(o, pw_ref[...], preferred_element_type=jnp.float32) + pb_ref[...]
        m = y.mean(-1, keepdims=True)
        v = ((y - m) ** 2).mean(-1, keepdims=True)
        y = (y - m) * lax.rsqrt(v + 1e-5) * g_ref[...] + lb_ref[...]
        o_ref[...] = a_ref[...] + y

    return pl.pallas_call(
        body, grid=(NR // NBT,),
        in_specs=[
            pl.BlockSpec((NC, NBT, H), lambda i: (0, i, 0)),
            pl.BlockSpec((NC, NBT, HEADS), lambda i: (0, i, 0)),
            pl.BlockSpec((NBT, H), lambda i: (i, 0)),
            pl.BlockSpec((H,), lambda i: (0,)),
            pl.BlockSpec((H, H), lambda i: (0, 0)),
            pl.BlockSpec((H,), lambda i: (0,)),
            pl.BlockSpec((H,), lambda i: (0,)),
            pl.BlockSpec((H,), lambda i: (0,)),
        ],
        out_specs=pl.BlockSpec((NBT, H), lambda i: (i, 0)),
        out_shape=jax.ShapeDtypeStruct((NR, H), jnp.float32),
    )(agg2, s2r, attn, bcat, projW, projb, lng, lnb)


def _gcn_pre_tc(xp, W0, deg2):
    def body(x_ref, w_ref, d_ref, hp_ref):
        deg = d_ref[0] + d_ref[1]
        dinv = jnp.where(deg > 0, lax.rsqrt(deg), 0.0)
        hp_ref[...] = dinv[:, None] * jnp.dot(x_ref[...], w_ref[...],
                                              preferred_element_type=jnp.float32)

    return pl.pallas_call(
        body, grid=(NR // NBT,),
        in_specs=[
            pl.BlockSpec((NBT, H), lambda i: (i, 0)),
            pl.BlockSpec((H, H), lambda i: (0, 0)),
            pl.BlockSpec((NC, NBT), lambda i: (0, i)),
        ],
        out_specs=pl.BlockSpec((NBT, H), lambda i: (i, 0)),
        out_shape=jax.ShapeDtypeStruct((NR, H), jnp.float32),
    )(xp, W0, deg2)


def _gcn_step_tc(agg2, deg2, trad, bvec, Wnext):
    def body(ag_ref, d_ref, t_ref, b_ref, wn_ref, tn_ref, hp_ref):
        deg = d_ref[0] + d_ref[1]
        dinv = jnp.where(deg > 0, lax.rsqrt(deg), 0.0)
        agg = dinv[:, None] * (ag_ref[0] + ag_ref[1]) + b_ref[...][None, :]
        tn = t_ref[...] + jnp.maximum(agg, 0.0)
        tn_ref[...] = tn
        hp_ref[...] = dinv[:, None] * jnp.dot(tn, wn_ref[...],
                                              preferred_element_type=jnp.float32)

    return pl.pallas_call(
        body, grid=(NR // NBT,),
        in_specs=[
            pl.BlockSpec((NC, NBT, H), lambda i: (0, i, 0)),
            pl.BlockSpec((NC, NBT), lambda i: (0, i)),
            pl.BlockSpec((NBT, H), lambda i: (i, 0)),
            pl.BlockSpec((H,), lambda i: (0,)),
            pl.BlockSpec((H, H), lambda i: (0, 0)),
        ],
        out_specs=[
            pl.BlockSpec((NBT, H), lambda i: (i, 0)),
            pl.BlockSpec((NBT, H), lambda i: (i, 0)),
        ],
        out_shape=[
            jax.ShapeDtypeStruct((NR, H), jnp.float32),
            jax.ShapeDtypeStruct((NR, H), jnp.float32),
        ],
    )(agg2, deg2, trad, bvec, Wnext)


def _pool_tc(attn, trad):
    def body(a_ref, t_ref, o_ref):
        cb = a_ref[...] + t_ref[...]

        @pl.when(pl.program_id(0) == 0)
        def _():
            o_ref[...] = jnp.full((8, H), -jnp.inf, jnp.float32)
            o_ref[0:1, :] = jnp.zeros((1, H), jnp.float32)

        o_ref[0:1, :] = o_ref[0:1, :] + jnp.sum(cb, axis=0, keepdims=True)
        o_ref[1:2, :] = jnp.maximum(o_ref[1:2, :], jnp.max(cb, axis=0, keepdims=True))

    return pl.pallas_call(
        body, grid=(N // PB,),
        in_specs=[
            pl.BlockSpec((PB, H), lambda i: (i, 0)),
            pl.BlockSpec((PB, H), lambda i: (i, 0)),
        ],
        out_specs=pl.BlockSpec((8, H), lambda i: (0, 0)),
        out_shape=jax.ShapeDtypeStruct((8, H), jnp.float32),
    )(attn, trad)


def _heads_tc(pool, p):
    def body(pool_ref, dw1, db1, dw2, db2, dw3, db3,
             vw1, vb1, vw2, vb2, vw3, vb3,
             tw1, tb1, tw2, tb2, sw1, sb1, sw2, sb2,
             dec_ref, val_ref, tmp_ref, safe_ref):
        mean = pool_ref[0:1, :] * (1.0 / N)
        mx = pool_ref[1:2, :]
        g = jnp.concatenate([mean, mx], axis=1)

        def mm(a, w_ref, b_ref):
            return jnp.dot(a, w_ref[...], preferred_element_type=jnp.float32) + b_ref[...]

        h1 = jnp.maximum(mm(g, dw1, db1), 0.0)
        h2 = jnp.maximum(mm(h1, dw2, db2), 0.0)
        dec_ref[...] = jax.nn.sigmoid(mm(h2, dw3, db3))
        h1 = jnp.maximum(mm(g, vw1, vb1), 0.0)
        h2 = jnp.maximum(mm(h1, vw2, vb2), 0.0)
        val_ref[...] = mm(h2, vw3, vb3)
        h1 = jnp.maximum(mm(g, tw1, tb1), 0.0)
        tmp_ref[...] = mm(h1, tw2, tb2)
        h1 = jnp.maximum(mm(g, sw1, sb1), 0.0)
        safe_ref[...] = jax.nn.sigmoid(mm(h1, sw2, sb2))

    names = ['dec_W1', 'dec_b1', 'dec_W2', 'dec_b2', 'dec_W3', 'dec_b3',
             'val_W1', 'val_b1', 'val_W2', 'val_b2', 'val_W3', 'val_b3',
             'tmp_W1', 'tmp_b1', 'tmp_W2', 'tmp_b2',
             'safe_W1', 'safe_b1', 'safe_W2', 'safe_b2']
    args = [pool] + [p[n] for n in names]
    return pl.pallas_call(
        body,
        out_shape=[
            jax.ShapeDtypeStruct((1, 4), jnp.float32),
            jax.ShapeDtypeStruct((1, 1), jnp.float32),
            jax.ShapeDtypeStruct((1, 3), jnp.float32),
            jax.ShapeDtypeStruct((1, 1), jnp.float32),
        ],
    )(*args)


def kernel(node_features, edge_index, params):
    p = params
    loop = jnp.arange(N, dtype=edge_index.dtype)
    pad = jnp.full((NE_PAD - NE_TOT,), DUMMY, dtype=edge_index.dtype)
    srcw = jnp.concatenate([edge_index[0], loop, pad])
    dstw = jnp.concatenate([edge_index[1], loop, pad])

    deg2 = _deg_kernel()(dstw)

    nfp = jnp.zeros((NR, DF), jnp.float32).at[:N].set(node_features)
    x = _enc_tc(nfp, p)

    attn = x
    for i in range(L):
        Wcat = jnp.moveaxis(p['gat_W'][i], 0, 1).reshape(H, H)
        h, asn, adn = _gat_pre_tc(attn, Wcat, p['gat_asrc'][i], p['gat_adst'][i])
        M = jnp.max(asn, axis=0) + jnp.max(adn, axis=0)
        M = jnp.where(M >= 0, M, 0.2 * M)
        Mp = jnp.zeros((LANES,), jnp.float32).at[:HEADS].set(M)
        ex, s2 = _att_kernel()(asn.reshape(-1), adn.reshape(-1), Mp, srcw, dstw)
        agg2 = _edge_kernel(True)(h, srcw, dstw, ex)
        attn = _gat_post_tc(agg2, s2.reshape(NC, NR, HEADS), attn,
                            p['gat_b'][i].reshape(H), p['proj_W'][i],
                            p['proj_b'][i], p['ln_g'][i], p['ln_b'][i])

    trad = x
    hp = _gcn_pre_tc(x, p['gcn_W'][0], deg2)
    for i in range(L):
        agg2 = _edge_kernel(False)(hp, srcw, dstw)
        Wnext = p['gcn_W'][(i + 1) % L]
        trad, hp = _gcn_step_tc(agg2, deg2, trad, p['gcn_b'][i], Wnext)

    pool = _pool_tc(attn, trad)
    dec, val, temp, safe = _heads_tc(pool, p)
    return (dec.reshape(4), val.reshape(1), temp.reshape(3), safe.reshape(1))
